# Initial kernel scaffold; baseline (speedup 1.0000x reference)
import functools

import jax
import jax.numpy as jnp
from jax import lax
from jax.experimental import pallas as pl
from jax.experimental.pallas import tpu as pltpu
from jax.experimental.pallas import tpu_sc as plsc

_EMBED = 32
_BLOCK_ROWS = 2000


def _body(n_blocks, num_cores, xf, w0f, w1f, w2f, of,
          xv, tab, w0v, w1v, w2v, ov):
    wid = lax.axis_index("s") * num_cores + lax.axis_index("c")

    pltpu.sync_copy(w0f, w0v)
    pltpu.sync_copy(w1f, w1v)
    pltpu.sync_copy(w2f, w2v)

    for a in range(5):
        r0l = w0v[pl.ds(a * 32, 16)]
        r0h = w0v[pl.ds(a * 32 + 16, 16)]
        for b in range(5):
            r1l = r0l + w1v[pl.ds(b * 32, 16)]
            r1h = r0h + w1v[pl.ds(b * 32 + 16, 16)]
            for c in range(5):
                o = (a * 25 + b * 5 + c) * 32
                tab[pl.ds(o, 16)] = r1l + w2v[pl.ds(c * 32, 16)]
                tab[pl.ds(o + 16, 16)] = r1h + w2v[pl.ds(c * 32 + 16, 16)]

    lanes = lax.iota(jnp.int32, 16)
    i3 = lanes * 3
    ost = lanes * 32
    rows_per_worker = n_blocks * _BLOCK_ROWS
    base_row = wid * rows_per_worker

    for blk in range(n_blocks):
        rb = base_row + blk * _BLOCK_ROWS
        pltpu.sync_copy(xf.at[pl.ds(rb * 3, _BLOCK_ROWS * 3)], xv)

        def grp(g, carry):
            idx0 = i3 + g * 48
            x0 = plsc.load_gather(xv, [idx0])
            x1 = plsc.load_gather(xv, [idx0 + 1])
            x2 = plsc.load_gather(xv, [idx0 + 2])
            addr = (x0 * 25 + x1 * 5 + x2) * 32
            op = ost + g * 512
            for c in range(_EMBED):
                v = plsc.load_gather(tab, [addr + c])
                plsc.store_scatter(ov, [op + c], v)
            return carry

        lax.fori_loop(0, _BLOCK_ROWS // 16, grp, 0)
        pltpu.sync_copy(ov, of.at[pl.ds(rb * _EMBED, _BLOCK_ROWS * _EMBED)])


def kernel(x, w0, w1, w2):
    e = x.shape[0]
    info = plsc.get_sparse_core_info()
    nw = info.num_cores * info.num_subcores
    assert e % (nw * _BLOCK_ROWS) == 0
    n_blocks = e // (nw * _BLOCK_ROWS)

    mesh = plsc.VectorSubcoreMesh(core_axis_name="c", subcore_axis_name="s")
    k = pl.kernel(
        functools.partial(_body, n_blocks, info.num_cores),
        out_type=jax.ShapeDtypeStruct((e * _EMBED,), jnp.float32),
        mesh=mesh,
        scratch_types=[
            pltpu.VMEM((_BLOCK_ROWS * 3,), jnp.int32),
            pltpu.VMEM((125 * _EMBED,), jnp.float32),
            pltpu.VMEM((w0.size,), jnp.float32),
            pltpu.VMEM((w1.size,), jnp.float32),
            pltpu.VMEM((w2.size,), jnp.float32),
            pltpu.VMEM((_BLOCK_ROWS * _EMBED,), jnp.float32),
        ],
    )
    out = k(x.reshape(-1), w0.reshape(-1), w1.reshape(-1), w2.reshape(-1))
    return out.reshape(e, _EMBED)


# speedup vs baseline: 1.8156x; 1.8156x over previous
import functools

import jax
import jax.numpy as jnp
from jax import lax
from jax.experimental import pallas as pl
from jax.experimental.pallas import tpu as pltpu
from jax.experimental.pallas import tpu_sc as plsc

_EMBED = 32
_BLOCK_ROWS = 2000
_GATHER_DN = lax.GatherDimensionNumbers(
    offset_dims=(), collapsed_slice_dims=(0,), start_index_map=(0,))


def _reg_gather(v, idx):
    return lax.gather(v, idx[:, None], _GATHER_DN, slice_sizes=(1,),
                      mode=lax.GatherScatterMode.PROMISE_IN_BOUNDS)


def _body(n_blocks, num_cores, xf, w0f, w1f, w2f, of,
          xv, tabl, tabh, w0v, w1v, w2v, ov):
    wid = lax.axis_index("s") * num_cores + lax.axis_index("c")

    pltpu.sync_copy(w0f, w0v)
    pltpu.sync_copy(w1f, w1v)
    pltpu.sync_copy(w2f, w2v)

    for a in range(5):
        r0l = w0v[pl.ds(a * 32, 16)]
        r0h = w0v[pl.ds(a * 32 + 16, 16)]
        for b in range(5):
            r1l = r0l + w1v[pl.ds(b * 32, 16)]
            r1h = r0h + w1v[pl.ds(b * 32 + 16, 16)]
            for c in range(5):
                o = (a * 25 + b * 5 + c) * 16
                tabl[pl.ds(o, 16)] = r1l + w2v[pl.ds(c * 32, 16)]
                tabh[pl.ds(o, 16)] = r1h + w2v[pl.ds(c * 32 + 16, 16)]

    lanes = lax.iota(jnp.int32, 16)
    qs, m0s, m1s = [], [], []
    for k in range(3):
        p = lanes * 3 + k
        qs.append(p & 15)
        r = lax.shift_right_logical(p, 4)
        m0s.append(r == 0)
        m1s.append(r == 1)

    rows_per_worker = n_blocks * _BLOCK_ROWS
    base_row = wid * rows_per_worker

    for blk in range(n_blocks):
        rb = base_row + blk * _BLOCK_ROWS
        pltpu.sync_copy(xf.at[pl.ds(rb * 3, _BLOCK_ROWS * 3)], xv)

        def grp(g, carry):
            b = g * 48
            va = xv[pl.ds(b, 16)]
            vb = xv[pl.ds(b + 16, 16)]
            vc = xv[pl.ds(b + 32, 16)]
            comp = []
            for k in range(3):
                g0 = _reg_gather(va, qs[k])
                g1 = _reg_gather(vb, qs[k])
                g2 = _reg_gather(vc, qs[k])
                comp.append(jnp.where(m0s[k], g0, jnp.where(m1s[k], g1, g2)))
            cv = (comp[0] * 25 + comp[1] * 5 + comp[2]) * 16
            obase = g * (16 * _EMBED)
            for l in range(16):
                s = cv[l]
                o = obase + l * _EMBED
                ov[pl.ds(o, 16)] = tabl[pl.ds(s, 16)]
                ov[pl.ds(o + 16, 16)] = tabh[pl.ds(s, 16)]
            return carry

        lax.fori_loop(0, _BLOCK_ROWS // 16, grp, 0)
        pltpu.sync_copy(ov, of.at[pl.ds(rb * _EMBED, _BLOCK_ROWS * _EMBED)])


def kernel(x, w0, w1, w2):
    e = x.shape[0]
    info = plsc.get_sparse_core_info()
    nw = info.num_cores * info.num_subcores
    assert e % (nw * _BLOCK_ROWS) == 0
    n_blocks = e // (nw * _BLOCK_ROWS)

    mesh = plsc.VectorSubcoreMesh(core_axis_name="c", subcore_axis_name="s")
    k = pl.kernel(
        functools.partial(_body, n_blocks, info.num_cores),
        out_type=jax.ShapeDtypeStruct((e * _EMBED,), jnp.float32),
        mesh=mesh,
        scratch_types=[
            pltpu.VMEM((_BLOCK_ROWS * 3,), jnp.int32),
            pltpu.VMEM((125 * 16,), jnp.float32),
            pltpu.VMEM((125 * 16,), jnp.float32),
            pltpu.VMEM((w0.size,), jnp.float32),
            pltpu.VMEM((w1.size,), jnp.float32),
            pltpu.VMEM((w2.size,), jnp.float32),
            pltpu.VMEM((_BLOCK_ROWS * _EMBED,), jnp.float32),
        ],
    )
    out = k(x.reshape(-1), w0.reshape(-1), w1.reshape(-1), w2.reshape(-1))
    return out.reshape(e, _EMBED)


# speedup vs baseline: 9.0199x; 4.9679x over previous
import functools

import jax
import jax.numpy as jnp
from jax import lax
from jax.experimental import pallas as pl
from jax.experimental.pallas import tpu as pltpu
from jax.experimental.pallas import tpu_sc as plsc

_EMBED = 32
_TILE = 128
_CHUNK_TILES = 16


def _body(num_workers, num_cores, xt, w0f, w1f, w2f, of,
          xv, tabl, tabh, w0v, w1v, w2v, ov):
    wid = lax.axis_index("s") * num_cores + lax.axis_index("c")

    pltpu.sync_copy(w0f, w0v)
    pltpu.sync_copy(w1f, w1v)
    pltpu.sync_copy(w2f, w2v)

    for a in range(5):
        r0l = w0v[pl.ds(a * 32, 16)]
        r0h = w0v[pl.ds(a * 32 + 16, 16)]
        for b in range(5):
            r1l = r0l + w1v[pl.ds(b * 32, 16)]
            r1h = r0h + w1v[pl.ds(b * 32 + 16, 16)]
            for c in range(5):
                o = (a * 25 + b * 5 + c) * 16
                tabl[pl.ds(o, 16)] = r1l + w2v[pl.ds(c * 32, 16)]
                tabh[pl.ds(o, 16)] = r1h + w2v[pl.ds(c * 32 + 16, 16)]

    n_tiles = xt.shape[1] // _TILE
    base_cnt = n_tiles // num_workers
    n_extra = n_tiles % num_workers
    cnt = base_cnt + jnp.where(wid < n_extra, 1, 0)
    start = base_cnt * wid + jnp.minimum(wid, n_extra)

    def run_chunk(tile_base, tiles):
        rows = tiles * _TILE
        pltpu.sync_copy(xt.at[:, pl.ds(tile_base * _TILE, rows)],
                        xv.at[:, pl.ds(0, rows)])

        def grp(g, carry):
            b = g * 16
            x0 = xv[0, pl.ds(b, 16)]
            x1 = xv[1, pl.ds(b, 16)]
            x2 = xv[2, pl.ds(b, 16)]
            cv = (x0 * 25 + x1 * 5 + x2) * 16
            obase = g * (16 * _EMBED)
            for l in range(16):
                s = cv[l]
                o = obase + l * _EMBED
                ov[pl.ds(o, 16)] = tabl[pl.ds(s, 16)]
                ov[pl.ds(o + 16, 16)] = tabh[pl.ds(s, 16)]
            return carry

        lax.fori_loop(0, rows // 16, grp, 0)
        pltpu.sync_copy(ov.at[pl.ds(0, rows * _EMBED)],
                        of.at[pl.ds(tile_base * _TILE * _EMBED, rows * _EMBED)])

    n_full = base_cnt // _CHUNK_TILES

    def full_chunk(i, carry):
        run_chunk(start + i * _CHUNK_TILES, _CHUNK_TILES)
        return carry

    lax.fori_loop(0, n_full, full_chunk, 0)

    def rem_tile(j, carry):
        run_chunk(start + n_full * _CHUNK_TILES + j, 1)
        return carry

    lax.fori_loop(0, cnt - n_full * _CHUNK_TILES, rem_tile, 0)


def kernel(x, w0, w1, w2):
    e = x.shape[0]
    info = plsc.get_sparse_core_info()
    nw = info.num_cores * info.num_subcores
    assert e % _TILE == 0

    mesh = plsc.VectorSubcoreMesh(core_axis_name="c", subcore_axis_name="s")
    rows_chunk = _CHUNK_TILES * _TILE
    k = pl.kernel(
        functools.partial(_body, nw, info.num_cores),
        out_type=jax.ShapeDtypeStruct((e * _EMBED,), jnp.float32),
        mesh=mesh,
        scratch_types=[
            pltpu.VMEM((3, rows_chunk), jnp.int32),
            pltpu.VMEM((125 * 16,), jnp.float32),
            pltpu.VMEM((125 * 16,), jnp.float32),
            pltpu.VMEM((w0.size,), jnp.float32),
            pltpu.VMEM((w1.size,), jnp.float32),
            pltpu.VMEM((w2.size,), jnp.float32),
            pltpu.VMEM((rows_chunk * _EMBED,), jnp.float32),
        ],
    )
    out = k(x.T, w0.reshape(-1), w1.reshape(-1), w2.reshape(-1))
    return out.reshape(e, _EMBED)


# speedup vs baseline: 43.0722x; 4.7752x over previous
import functools

import jax
import jax.numpy as jnp
from jax import lax
from jax.experimental import pallas as pl
from jax.experimental.pallas import tpu as pltpu
from jax.experimental.pallas import tpu_sc as plsc

_EMBED = 32
_TILE = 128
_CHUNK_TILES = 16
_GATHER_DN = lax.GatherDimensionNumbers(
    offset_dims=(), collapsed_slice_dims=(0,), start_index_map=(0,))


def _reg_gather(v, idx):
    return lax.gather(v, idx[:, None], _GATHER_DN, slice_sizes=(1,),
                      mode=lax.GatherScatterMode.PROMISE_IN_BOUNDS)


def _body(num_workers, num_cores, xt, w0t, w1t, w2t, of,
          xv, w0v, w1v, w2v, qv):
    wid = lax.axis_index("s") * num_cores + lax.axis_index("c")

    pltpu.sync_copy(w0t, w0v)
    pltpu.sync_copy(w1t, w1v)
    pltpu.sync_copy(w2t, w2v)

    n_tiles = xt.shape[1] // _TILE
    base_cnt = n_tiles // num_workers
    n_extra = n_tiles % num_workers
    cnt = base_cnt + jnp.where(wid < n_extra, 1, 0)
    start = base_cnt * wid + jnp.minimum(wid, n_extra)

    def run_chunk(tile_base, tiles):
        rows = tiles * _TILE
        col = tile_base * _TILE
        pltpu.sync_copy(xt.at[:, pl.ds(col, rows)], xv.at[:, pl.ds(0, rows)])

        for dh in range(2):
            t0 = [w0v[pl.ds((dh * 16 + i) * 16, 16)] for i in range(16)]
            t1 = [w1v[pl.ds((dh * 16 + i) * 16, 16)] for i in range(16)]
            t2 = [w2v[pl.ds((dh * 16 + i) * 16, 16)] for i in range(16)]

            def grp(g, carry):
                b = g * 16
                x0 = xv[0, pl.ds(b, 16)]
                x1 = xv[1, pl.ds(b, 16)]
                x2 = xv[2, pl.ds(b, 16)]
                for i in range(16):
                    v = (_reg_gather(t0[i], x0) + _reg_gather(t1[i], x1)
                         + _reg_gather(t2[i], x2))
                    qv[dh * 16 + i, pl.ds(b, 16)] = v
                return carry

            lax.fori_loop(0, rows // 16, grp, 0)

        pltpu.sync_copy(qv.at[:, pl.ds(0, rows)], of.at[:, pl.ds(col, rows)])

    n_full = base_cnt // _CHUNK_TILES

    def full_chunk(i, carry):
        run_chunk(start + i * _CHUNK_TILES, _CHUNK_TILES)
        return carry

    lax.fori_loop(0, n_full, full_chunk, 0)

    def rem_tile(j, carry):
        run_chunk(start + n_full * _CHUNK_TILES + j, 1)
        return carry

    lax.fori_loop(0, cnt - n_full * _CHUNK_TILES, rem_tile, 0)


def kernel(x, w0, w1, w2):
    e = x.shape[0]
    info = plsc.get_sparse_core_info()
    nw = info.num_cores * info.num_subcores
    assert e % _TILE == 0

    def tcols(w):
        return jnp.pad(w.T, ((0, 0), (0, 16 - w.shape[0]))).reshape(-1)

    mesh = plsc.VectorSubcoreMesh(core_axis_name="c", subcore_axis_name="s")
    rows_chunk = _CHUNK_TILES * _TILE
    k = pl.kernel(
        functools.partial(_body, nw, info.num_cores),
        out_type=jax.ShapeDtypeStruct((_EMBED, e), jnp.float32),
        mesh=mesh,
        scratch_types=[
            pltpu.VMEM((3, rows_chunk), jnp.int32),
            pltpu.VMEM((_EMBED * 16,), jnp.float32),
            pltpu.VMEM((_EMBED * 16,), jnp.float32),
            pltpu.VMEM((_EMBED * 16,), jnp.float32),
            pltpu.VMEM((_EMBED, rows_chunk), jnp.float32),
        ],
    )
    out = k(x.T, tcols(w0), tcols(w1), tcols(w2))
    return out.T


# speedup vs baseline: 62.6195x; 1.4538x over previous
import functools

import jax
import jax.numpy as jnp
from jax import lax
from jax.experimental import pallas as pl
from jax.experimental.pallas import tpu as pltpu
from jax.experimental.pallas import tpu_sc as plsc

_EMBED = 32
_TILE = 128
_CHUNK_TILES = 8
_GATHER_DN = lax.GatherDimensionNumbers(
    offset_dims=(), collapsed_slice_dims=(0,), start_index_map=(0,))


def _reg_gather(v, idx):
    return lax.gather(v, idx[:, None], _GATHER_DN, slice_sizes=(1,),
                      mode=lax.GatherScatterMode.PROMISE_IN_BOUNDS)


def _body(num_workers, num_cores, xt, w0t, w1t, w2t, of,
          xv0, xv1, qv0, qv1, w0v, w1v, w2v,
          sin0, sin1, sout0, sout1):
    wid = lax.axis_index("s") * num_cores + lax.axis_index("c")

    pltpu.sync_copy(w0t, w0v)
    pltpu.sync_copy(w1t, w1v)
    pltpu.sync_copy(w2t, w2v)

    xvs, qvs = (xv0, xv1), (qv0, qv1)
    sins, souts = (sin0, sin1), (sout0, sout1)
    rows = _CHUNK_TILES * _TILE

    n_tiles = xt.shape[1] // _TILE
    base_cnt = n_tiles // num_workers
    n_extra = n_tiles % num_workers
    cnt = base_cnt + jnp.where(wid < n_extra, 1, 0)
    start = base_cnt * wid + jnp.minimum(wid, n_extra)
    n_full = base_cnt // _CHUNK_TILES
    assert n_full % 2 == 0

    def in_start(c, b):
        col = (start + c * _CHUNK_TILES) * _TILE
        pltpu.async_copy(xt.at[:, pl.ds(col, rows)], xvs[b], sins[b])

    def in_wait(b):
        pltpu.make_async_copy(xt.at[:, pl.ds(0, rows)], xvs[b], sins[b]).wait()

    def out_start(c, b):
        col = (start + c * _CHUNK_TILES) * _TILE
        pltpu.async_copy(qvs[b], of.at[:, pl.ds(col, rows)], souts[b])

    def out_wait(b):
        pltpu.make_async_copy(qvs[b], of.at[:, pl.ds(0, rows)], souts[b]).wait()

    def compute(xv, qv, n_rows):
        for dh in range(2):
            t0 = [w0v[pl.ds((dh * 16 + i) * 16, 16)] for i in range(16)]
            t1 = [w1v[pl.ds((dh * 16 + i) * 16, 16)] for i in range(16)]
            t2 = [w2v[pl.ds((dh * 16 + i) * 16, 16)] for i in range(16)]

            def grp(g, carry):
                b = g * 16
                x0 = xv[0, pl.ds(b, 16)]
                x1 = xv[1, pl.ds(b, 16)]
                x2 = xv[2, pl.ds(b, 16)]
                for i in range(16):
                    v = (_reg_gather(t0[i], x0) + _reg_gather(t1[i], x1)
                         + _reg_gather(t2[i], x2))
                    qv[dh * 16 + i, pl.ds(b, 16)] = v
                return carry

            lax.fori_loop(0, n_rows // 16, grp, 0)

    in_start(0, 0)
    in_start(1, 1)

    def pair(p, carry):
        for b in range(2):
            c = 2 * p + b

            @pl.when(p >= 1)
            def _():
                out_wait(b)

            in_wait(b)
            compute(xvs[b], qvs[b], rows)
            out_start(c, b)

            @pl.when(c + 2 < n_full)
            def _():
                in_start(c + 2, b)
        return carry

    lax.fori_loop(0, n_full // 2, pair, 0)
    out_wait(0)
    out_wait(1)

    def rem_tile(j, carry):
        tb = start + n_full * _CHUNK_TILES + j
        col = tb * _TILE
        pltpu.sync_copy(xt.at[:, pl.ds(col, _TILE)],
                        xv0.at[:, pl.ds(0, _TILE)])
        compute(xv0, qv0, _TILE)
        pltpu.sync_copy(qv0.at[:, pl.ds(0, _TILE)],
                        of.at[:, pl.ds(col, _TILE)])
        return carry

    lax.fori_loop(0, cnt - n_full * _CHUNK_TILES, rem_tile, 0)


def kernel(x, w0, w1, w2):
    e = x.shape[0]
    info = plsc.get_sparse_core_info()
    nw = info.num_cores * info.num_subcores
    assert e % _TILE == 0

    def tcols(w):
        return jnp.pad(w.T, ((0, 0), (0, 16 - w.shape[0]))).reshape(-1)

    mesh = plsc.VectorSubcoreMesh(core_axis_name="c", subcore_axis_name="s")
    rows_chunk = _CHUNK_TILES * _TILE
    k = pl.kernel(
        functools.partial(_body, nw, info.num_cores),
        out_type=jax.ShapeDtypeStruct((_EMBED, e), jnp.float32),
        mesh=mesh,
        scratch_types=[
            pltpu.VMEM((3, rows_chunk), jnp.int32),
            pltpu.VMEM((3, rows_chunk), jnp.int32),
            pltpu.VMEM((_EMBED, rows_chunk), jnp.float32),
            pltpu.VMEM((_EMBED, rows_chunk), jnp.float32),
            pltpu.VMEM((_EMBED * 16,), jnp.float32),
            pltpu.VMEM((_EMBED * 16,), jnp.float32),
            pltpu.VMEM((_EMBED * 16,), jnp.float32),
            pltpu.SemaphoreType.DMA,
            pltpu.SemaphoreType.DMA,
            pltpu.SemaphoreType.DMA,
            pltpu.SemaphoreType.DMA,
        ],
    )
    out = k(x.T, tcols(w0), tcols(w1), tcols(w2))
    return out.T


# speedup vs baseline: 106.5376x; 1.7013x over previous
import functools

import jax
import jax.numpy as jnp
from jax import lax
from jax.experimental import pallas as pl
from jax.experimental.pallas import tpu as pltpu
from jax.experimental.pallas import tpu_sc as plsc

_EMBED = 32
_TILE = 128
_CHUNK_TILES = 8
_GATHER_DN = lax.GatherDimensionNumbers(
    offset_dims=(), collapsed_slice_dims=(0,), start_index_map=(0,))


def _reg_gather(v, idx):
    return lax.gather(v, idx[:, None], _GATHER_DN, slice_sizes=(1,),
                      mode=lax.GatherScatterMode.PROMISE_IN_BOUNDS)


def _body(num_workers, num_cores, xt, w0t, w1t, w2t, of,
          xv0, xv1, qv0, qv1, w0v, w1v, w2v,
          sin0, sin1, sout0, sout1):
    wid = lax.axis_index("s") * num_cores + lax.axis_index("c")

    pltpu.sync_copy(w0t, w0v)
    pltpu.sync_copy(w1t, w1v)
    pltpu.sync_copy(w2t, w2v)

    xvs, qvs = (xv0, xv1), (qv0, qv1)
    sins, souts = (sin0, sin1), (sout0, sout1)
    rows = _CHUNK_TILES * _TILE

    n_tiles = xt.shape[1] // _TILE
    base_cnt = n_tiles // num_workers
    n_extra = n_tiles % num_workers
    cnt = base_cnt + jnp.where(wid < n_extra, 1, 0)
    start = base_cnt * wid + jnp.minimum(wid, n_extra)
    n_full = base_cnt // _CHUNK_TILES
    assert n_full % 2 == 0

    def in_start(c, b):
        col = (start + c * _CHUNK_TILES) * _TILE
        pltpu.async_copy(xt.at[:, pl.ds(col, rows)], xvs[b], sins[b])

    def in_wait(b):
        pltpu.make_async_copy(xt.at[:, pl.ds(0, rows)], xvs[b], sins[b]).wait()

    def out_start(c, b):
        col = (start + c * _CHUNK_TILES) * _TILE
        pltpu.async_copy(qvs[b], of.at[:, pl.ds(col, rows)], souts[b])

    def out_wait(b):
        pltpu.make_async_copy(qvs[b], of.at[:, pl.ds(0, rows)], souts[b]).wait()

    def compute(xv, qv, n_rows):
        for dh in range(2):
            t0 = [w0v[pl.ds((dh * 16 + i) * 16, 16)] for i in range(16)]
            t1 = [w1v[pl.ds((dh * 16 + i) * 16, 16)] for i in range(16)]
            t2 = [w2v[pl.ds((dh * 16 + i) * 16, 16)] for i in range(16)]

            @functools.partial(plsc.parallel_loop, 0, n_rows // 16, unroll=2)
            def grp(g):
                b = g * 16
                x0 = xv[0, pl.ds(b, 16)]
                x1 = xv[1, pl.ds(b, 16)]
                x2 = xv[2, pl.ds(b, 16)]
                for i in range(16):
                    v = (_reg_gather(t0[i], x0) + _reg_gather(t1[i], x1)
                         + _reg_gather(t2[i], x2))
                    qv[dh * 16 + i, pl.ds(b, 16)] = v

    in_start(0, 0)
    in_start(1, 1)

    def pair(p, carry):
        for b in range(2):
            c = 2 * p + b

            @pl.when(p >= 1)
            def _():
                out_wait(b)

            in_wait(b)
            compute(xvs[b], qvs[b], rows)
            out_start(c, b)

            @pl.when(c + 2 < n_full)
            def _():
                in_start(c + 2, b)
        return carry

    lax.fori_loop(0, n_full // 2, pair, 0)
    out_wait(0)
    out_wait(1)

    def rem_tile(j, carry):
        tb = start + n_full * _CHUNK_TILES + j
        col = tb * _TILE
        pltpu.sync_copy(xt.at[:, pl.ds(col, _TILE)],
                        xv0.at[:, pl.ds(0, _TILE)])
        compute(xv0, qv0, _TILE)
        pltpu.sync_copy(qv0.at[:, pl.ds(0, _TILE)],
                        of.at[:, pl.ds(col, _TILE)])
        return carry

    lax.fori_loop(0, cnt - n_full * _CHUNK_TILES, rem_tile, 0)


def kernel(x, w0, w1, w2):
    e = x.shape[0]
    info = plsc.get_sparse_core_info()
    nw = info.num_cores * info.num_subcores
    assert e % _TILE == 0

    def tcols(w):
        return jnp.pad(w.T, ((0, 0), (0, 16 - w.shape[0]))).reshape(-1)

    mesh = plsc.VectorSubcoreMesh(core_axis_name="c", subcore_axis_name="s")
    rows_chunk = _CHUNK_TILES * _TILE
    k = pl.kernel(
        functools.partial(_body, nw, info.num_cores),
        out_type=jax.ShapeDtypeStruct((_EMBED, e), jnp.float32),
        mesh=mesh,
        scratch_types=[
            pltpu.VMEM((3, rows_chunk), jnp.int32),
            pltpu.VMEM((3, rows_chunk), jnp.int32),
            pltpu.VMEM((_EMBED, rows_chunk), jnp.float32),
            pltpu.VMEM((_EMBED, rows_chunk), jnp.float32),
            pltpu.VMEM((_EMBED * 16,), jnp.float32),
            pltpu.VMEM((_EMBED * 16,), jnp.float32),
            pltpu.VMEM((_EMBED * 16,), jnp.float32),
            pltpu.SemaphoreType.DMA,
            pltpu.SemaphoreType.DMA,
            pltpu.SemaphoreType.DMA,
            pltpu.SemaphoreType.DMA,
        ],
    )
    out = k(x.T, tcols(w0), tcols(w1), tcols(w2))
    return out.T
